# baseline (speedup 1.0000x reference)
import functools

import jax
import jax.numpy as jnp
from jax import lax
from jax.experimental import pallas as pl
from jax.experimental.pallas import tpu as pltpu
from jax.experimental.pallas import tpu_sc as plsc

_NC = 2
_NS = 16
_NL = 16


def _standardize_tc(x):
    n = x.shape[0]

    def body(x_ref, o_ref):
        xv = x_ref[...]
        mu = jnp.mean(xv, axis=0, keepdims=True)
        var = jnp.sum((xv - mu) ** 2, axis=0, keepdims=True) / (n - 1)
        o_ref[...] = (xv - mu) / (jnp.sqrt(var) + 1e-6)

    return pl.pallas_call(
        body, out_shape=jax.ShapeDtypeStruct(x.shape, x.dtype))(x)


def _mm3_tc(h, w1, b1, w2, w3, b3):
    n, d = h.shape
    o = w1.shape[1]

    def body(h_ref, w1_ref, b1_ref, w2_ref, w3_ref, b3_ref, a_ref, b_ref, c_ref):
        hv = h_ref[...]
        a_ref[...] = jnp.dot(hv, w1_ref[...], preferred_element_type=jnp.float32) + b1_ref[...]
        b_ref[...] = jnp.dot(hv, w2_ref[...], preferred_element_type=jnp.float32)
        c_ref[...] = jnp.dot(hv, w3_ref[...], preferred_element_type=jnp.float32) + b3_ref[...]

    outs = (jax.ShapeDtypeStruct((n, o), jnp.float32),) * 3
    return pl.pallas_call(body, out_shape=outs)(h, w1, b1, w2, w3, b3)


def _wsum_tc(wparts):
    n = wparts.shape[1]

    def body(p_ref, o_ref):
        o_ref[...] = jnp.sum(p_ref[...], axis=0)[:, None]

    return pl.pallas_call(
        body, out_shape=jax.ShapeDtypeStruct((n, 1), jnp.float32))(wparts)


def _combine_tc(parts, bm, cm, wdeg, g, be, final):
    n, d = bm.shape

    def body(p_ref, b_ref, c_ref, w_ref, g_ref, be_ref, o_ref):
        v = p_ref[0] + p_ref[1] - b_ref[...] * w_ref[...] + c_ref[...]
        if final:
            o_ref[...] = jax.nn.sigmoid(v)
        else:
            mu = jnp.mean(v, axis=1, keepdims=True)
            var = jnp.mean((v - mu) ** 2, axis=1, keepdims=True)
            y = (v - mu) / jnp.sqrt(var + 1e-5) * g_ref[...] + be_ref[...]
            o_ref[...] = jnp.where(y >= 0, y, 0.1 * y)

    return pl.pallas_call(
        body, out_shape=jax.ShapeDtypeStruct((n, d), jnp.float32))(
            parts, bm, cm, wdeg, g, be)


def _wdeg_sc(dst2, ew2, n):
    nw = _NC * _NS
    ew_per = dst2.shape[1]
    mesh = plsc.VectorSubcoreMesh(core_axis_name="c", subcore_axis_name="s")

    @functools.partial(
        pl.kernel,
        out_type=jax.ShapeDtypeStruct((nw, n), jnp.float32),
        mesh=mesh,
        compiler_params=pltpu.CompilerParams(needs_layout_passes=False),
        scratch_types=[
            pltpu.VMEM((ew_per,), jnp.int32),
            pltpu.VMEM((ew_per,), jnp.float32),
            pltpu.VMEM((n,), jnp.float32),
        ],
    )
    def k(dst_hbm, ew_hbm, out_hbm, dstv, ewv, wacc):
        c = lax.axis_index("c")
        s = lax.axis_index("s")
        wid = s * _NC + c

        def zero(i, carry):
            wacc[pl.ds(i * _NL, _NL)] = jnp.zeros((_NL,), jnp.float32)
            return carry

        lax.fori_loop(0, n // _NL, zero, 0)
        pltpu.sync_copy(dst_hbm.at[wid], dstv)
        pltpu.sync_copy(ew_hbm.at[wid], ewv)

        def accum(i, carry):
            dv = dstv[pl.ds(i * _NL, _NL)]
            wv = ewv[pl.ds(i * _NL, _NL)]
            plsc.addupdate_scatter(wacc, [dv], wv)
            return carry

        lax.fori_loop(0, ew_per // _NL, accum, 0)
        pltpu.sync_copy(wacc, out_hbm.at[wid])

    return k(dst2, ew2)


@functools.lru_cache(maxsize=None)
def _make_scatter_sc(n, d, nw, ngr, gch, ce):
    rps = n // _NS
    zr = 25
    while rps % zr:
        zr -= 1
    nq = (gch + 2 + 3) // 4
    mesh = plsc.VectorSubcoreMesh(core_axis_name="c", subcore_axis_name="s")

    @functools.partial(
        pl.kernel,
        out_type=jax.ShapeDtypeStruct((_NC, _NS, rps, d), jnp.float32),
        mesh=mesh,
        compiler_params=pltpu.CompilerParams(needs_layout_passes=False),
        scratch_types=[
            pltpu.VMEM((gch, ce), jnp.int32),
            pltpu.VMEM((gch, ce), jnp.int32),
            pltpu.VMEM((gch, ce), jnp.float32),
            pltpu.VMEM((ce, d), jnp.float32),
            pltpu.VMEM((ce, d), jnp.float32),
            pltpu.VMEM((ce, d), jnp.float32),
            pltpu.VMEM((ce, d), jnp.float32),
            pltpu.VMEM((zr, d), jnp.float32),
            pltpu.VMEM_SHARED((n, d), jnp.float32),
            pltpu.SemaphoreType.DMA,
            pltpu.SemaphoreType.DMA,
            pltpu.SemaphoreType.DMA,
            pltpu.SemaphoreType.DMA,
            pltpu.SemaphoreType.DMA,
            pltpu.SemaphoreType.DMA,
            pltpu.SemaphoreType.DMA,
            pltpu.SemaphoreType.DMA,
        ],
    )
    def k(a_hbm, src_hbm, dst_hbm, ew_hbm, out_hbm, srcg, dstg, ewg,
          r0, r1, r2, r3, zbuf, acc, g0, g1, g2, g3, s0, s1, s2, s3):
        rows = (r0, r1, r2, r3)
        gsem = (g0, g1, g2, g3)
        ssem = (s0, s1, s2, s3)
        c = lax.axis_index("c")
        s = lax.axis_index("s")
        wid = s * _NC + c

        def zrow(i, carry):
            for j in range(d // _NL):
                zbuf[i, pl.ds(j * _NL, _NL)] = jnp.zeros((_NL,), jnp.float32)
            return carry

        lax.fori_loop(0, zr, zrow, 0)
        for r in range(rps // zr):
            pltpu.sync_copy(zbuf, acc.at[pl.ds(s * rps + r * zr, zr)])
        plsc.subcore_barrier()

        def group(g, carry):
            pltpu.sync_copy(src_hbm.at[wid, g], srcg)
            pltpu.sync_copy(dst_hbm.at[wid, g], dstg)
            pltpu.sync_copy(ew_hbm.at[wid, g], ewg)

            def quad(q, c2):
                for b in range(4):
                    p = q * 4 + b
                    b2 = (b + 2) % 4

                    @pl.when(p < gch)
                    def _():
                        pltpu.async_copy(
                            a_hbm.at[srcg.at[p]], rows[b], gsem[b])

                    @pl.when(jnp.logical_and(p >= 2, p < gch + 2))
                    def _():
                        kk = p - 2
                        pltpu.make_async_copy(
                            a_hbm.at[srcg.at[kk]], rows[b2], gsem[b2]).wait()

                        def vec16(gg, c3):
                            wv = ewg[kk, pl.ds(gg * _NL, _NL)]
                            for t in range(_NL):
                                w = wv[t]
                                e = gg * _NL + t
                                for j in range(d // _NL):
                                    sl = pl.ds(j * _NL, _NL)
                                    rows[b2][e, sl] = rows[b2][e, sl] * w
                            return c3

                        lax.fori_loop(0, ce // _NL, vec16, 0)
                return c2

            lax.fori_loop(0, nq, quad, 0)
            return carry

        lax.fori_loop(0, ngr, group, 0)
        plsc.subcore_barrier()
        pltpu.sync_copy(acc.at[pl.ds(s * rps, rps)], out_hbm.at[c, s])

    return k


def _scatter_sc(a, src4, dst4, ew4):
    n, d = a.shape
    nw, ngr, gch, ce = src4.shape
    k = _make_scatter_sc(n, d, nw, ngr, gch, ce)
    return k(a, src4, dst4, ew4).reshape(_NC, n, d)


def kernel(x, edge_index, edge_attr, number_of_layers, W1_in, b1_in, W2_in,
           W3_in, b3_in, W1_hid, b1_hid, W2_hid, W3_hid, b3_hid, W1_out,
           b1_out, W2_out, W3_out, b3_out, g1, be1, g2, be2):
    n, d = x.shape
    e = edge_attr.shape[0]
    nw = _NC * _NS
    ew_per = e // nw
    ce = 32
    ngr = 5
    gch = -(-ew_per // (ce * ngr))
    epw = ngr * gch * ce

    def shard(a, fill):
        a = a.reshape(nw, ew_per)
        if epw != ew_per:
            a = jnp.pad(a, ((0, 0), (0, epw - ew_per)), constant_values=fill)
        return a.reshape(nw, ngr, gch, ce)

    src4 = shard(edge_index[0], 0)
    dst4 = shard(edge_index[1], 0)
    ew4 = shard(edge_attr, 0.0)

    z = _standardize_tc(x)
    wparts = _wdeg_sc(edge_index[1].reshape(nw, ew_per),
                      edge_attr.reshape(nw, ew_per), n)
    wdeg = _wsum_tc(wparts)

    def layer(h, w1, b1, w2, w3, b3, g, be, final):
        a, bm, cm = _mm3_tc(h, w1, b1, w2, w3, b3)
        parts = _scatter_sc(a, src4, dst4, ew4)
        return _combine_tc(parts, bm, cm, wdeg, g, be, final)

    h = layer(z, W1_in, b1_in, W2_in, W3_in, b3_in, g1, be1, False)
    h = lax.fori_loop(
        0, number_of_layers - 2,
        lambda _, hc: layer(hc, W1_hid, b1_hid, W2_hid, W3_hid, b3_hid,
                            g2, be2, False),
        h)
    return layer(h, W1_out, b1_out, W2_out, W3_out, b3_out, g1, be1, True)


# speedup vs baseline: 1.3970x; 1.3970x over previous
import functools

import jax
import jax.numpy as jnp
from jax import lax
from jax.experimental import pallas as pl
from jax.experimental.pallas import tpu as pltpu
from jax.experimental.pallas import tpu_sc as plsc

_NC = 2
_NS = 16
_NL = 16


def _standardize_tc(x):
    n = x.shape[0]

    def body(x_ref, o_ref):
        xv = x_ref[...]
        mu = jnp.mean(xv, axis=0, keepdims=True)
        var = jnp.sum((xv - mu) ** 2, axis=0, keepdims=True) / (n - 1)
        o_ref[...] = (xv - mu) / (jnp.sqrt(var) + 1e-6)

    return pl.pallas_call(
        body, out_shape=jax.ShapeDtypeStruct(x.shape, x.dtype))(x)


def _mm3_tc(h, w1, b1, w2, w3, b3):
    n, d = h.shape
    o = w1.shape[1]

    def body(h_ref, w1_ref, b1_ref, w2_ref, w3_ref, b3_ref, a_ref, b_ref, c_ref):
        hv = h_ref[...]
        a_ref[...] = jnp.dot(hv, w1_ref[...], preferred_element_type=jnp.float32) + b1_ref[...]
        b_ref[...] = jnp.dot(hv, w2_ref[...], preferred_element_type=jnp.float32)
        c_ref[...] = jnp.dot(hv, w3_ref[...], preferred_element_type=jnp.float32) + b3_ref[...]

    outs = (jax.ShapeDtypeStruct((n, o), jnp.float32),) * 3
    return pl.pallas_call(body, out_shape=outs)(h, w1, b1, w2, w3, b3)


def _wsum_tc(wparts):
    n = wparts.shape[1]

    def body(p_ref, o_ref):
        o_ref[...] = jnp.sum(p_ref[...], axis=0)[:, None]

    return pl.pallas_call(
        body, out_shape=jax.ShapeDtypeStruct((n, 1), jnp.float32))(wparts)


def _combine_tc(parts, bm, cm, wdeg, g, be, final):
    n, d = bm.shape

    def body(p_ref, b_ref, c_ref, w_ref, g_ref, be_ref, o_ref):
        v = p_ref[0] + p_ref[1] - b_ref[...] * w_ref[...] + c_ref[...]
        if final:
            o_ref[...] = jax.nn.sigmoid(v)
        else:
            mu = jnp.mean(v, axis=1, keepdims=True)
            var = jnp.mean((v - mu) ** 2, axis=1, keepdims=True)
            y = (v - mu) / jnp.sqrt(var + 1e-5) * g_ref[...] + be_ref[...]
            o_ref[...] = jnp.where(y >= 0, y, 0.1 * y)

    return pl.pallas_call(
        body, out_shape=jax.ShapeDtypeStruct((n, d), jnp.float32))(
            parts, bm, cm, wdeg, g, be)


def _wdeg_sc(dst2, ew2, n):
    nw = _NC * _NS
    ew_per = dst2.shape[1]
    mesh = plsc.VectorSubcoreMesh(core_axis_name="c", subcore_axis_name="s")

    @functools.partial(
        pl.kernel,
        out_type=jax.ShapeDtypeStruct((nw, n), jnp.float32),
        mesh=mesh,
        compiler_params=pltpu.CompilerParams(needs_layout_passes=False),
        scratch_types=[
            pltpu.VMEM((ew_per,), jnp.int32),
            pltpu.VMEM((ew_per,), jnp.float32),
            pltpu.VMEM((n,), jnp.float32),
        ],
    )
    def k(dst_hbm, ew_hbm, out_hbm, dstv, ewv, wacc):
        c = lax.axis_index("c")
        s = lax.axis_index("s")
        wid = s * _NC + c

        def zero(i, carry):
            wacc[pl.ds(i * _NL, _NL)] = jnp.zeros((_NL,), jnp.float32)
            return carry

        lax.fori_loop(0, n // _NL, zero, 0)
        pltpu.sync_copy(dst_hbm.at[wid], dstv)
        pltpu.sync_copy(ew_hbm.at[wid], ewv)

        def accum(i, carry):
            dv = dstv[pl.ds(i * _NL, _NL)]
            wv = ewv[pl.ds(i * _NL, _NL)]
            plsc.addupdate_scatter(wacc, [dv], wv)
            return carry

        lax.fori_loop(0, ew_per // _NL, accum, 0)
        pltpu.sync_copy(wacc, out_hbm.at[wid])

    return k(dst2, ew2)


@functools.lru_cache(maxsize=None)
def _make_scatter_sc(n, d, nw, ngr, gch, ce):
    rps = n // _NS
    zr = 25
    while rps % zr:
        zr -= 1
    nq = (gch + 1 + 1) // 2
    mesh = plsc.VectorSubcoreMesh(core_axis_name="c", subcore_axis_name="s")

    @functools.partial(
        pl.kernel,
        out_type=jax.ShapeDtypeStruct((_NC, _NS, rps, d), jnp.float32),
        mesh=mesh,
        compiler_params=pltpu.CompilerParams(needs_layout_passes=False),
        scratch_types=[
            pltpu.VMEM((gch, ce), jnp.int32),
            pltpu.VMEM((gch, ce), jnp.int32),
            pltpu.VMEM((gch, ce), jnp.float32),
            pltpu.VMEM((ce, d), jnp.float32),
            pltpu.VMEM((ce, d), jnp.float32),
            pltpu.VMEM((zr, d), jnp.float32),
            pltpu.VMEM_SHARED((n, d), jnp.float32),
            pltpu.SemaphoreType.DMA,
            pltpu.SemaphoreType.DMA,
            pltpu.SemaphoreType.DMA,
            pltpu.SemaphoreType.DMA,
        ],
    )
    def k(a_hbm, src_hbm, dst_hbm, ew_hbm, out_hbm, srcg, dstg, ewg,
          r0, r1, zbuf, acc, g0, g1, s0, s1):
        rows = (r0, r1)
        gsem = (g0, g1)
        ssem = (s0, s1)
        c = lax.axis_index("c")
        s = lax.axis_index("s")
        wid = s * _NC + c

        def zrow(i, carry):
            for j in range(d // _NL):
                zbuf[i, pl.ds(j * _NL, _NL)] = jnp.zeros((_NL,), jnp.float32)
            return carry

        lax.fori_loop(0, zr, zrow, 0)
        for r in range(rps // zr):
            pltpu.sync_copy(zbuf, acc.at[pl.ds(s * rps + r * zr, zr)])
        plsc.subcore_barrier()

        def group(g, carry):
            pltpu.sync_copy(src_hbm.at[wid, g], srcg)
            pltpu.sync_copy(dst_hbm.at[wid, g], dstg)
            pltpu.sync_copy(ew_hbm.at[wid, g], ewg)

            def pair(q, c2):
                for b in range(2):
                    p = q * 2 + b
                    b1 = (b + 1) % 2

                    @pl.when(jnp.logical_and(p >= 2, p < gch))
                    def _():
                        pltpu.make_async_copy(
                            rows[b], acc.at[dstg.at[p - 2]], ssem[b]).wait()

                    @pl.when(p < gch)
                    def _():
                        pltpu.async_copy(
                            a_hbm.at[srcg.at[p]], rows[b], gsem[b])

                    @pl.when(jnp.logical_and(p >= 1, p < gch + 1))
                    def _():
                        kk = p - 1
                        pltpu.make_async_copy(
                            a_hbm.at[srcg.at[kk]], rows[b1], gsem[b1]).wait()

                        def vec16(gg, c3):
                            wv = ewg[kk, pl.ds(gg * _NL, _NL)]
                            for t in range(_NL):
                                w = wv[t]
                                e = gg * _NL + t
                                for j in range(d // _NL):
                                    sl = pl.ds(j * _NL, _NL)
                                    rows[b1][e, sl] = rows[b1][e, sl] * w
                            return c3

                        lax.fori_loop(0, ce // _NL, vec16, 0)
                        pltpu.async_copy(
                            rows[b1], acc.at[dstg.at[kk]], ssem[b1], add=True)
                return c2

            lax.fori_loop(0, nq, pair, 0)
            for b in range(2):
                kk = gch - 2 + ((b - (gch - 2)) % 2)
                pltpu.make_async_copy(
                    rows[b], acc.at[dstg.at[kk]], ssem[b]).wait()
            return carry

        lax.fori_loop(0, ngr, group, 0)
        plsc.subcore_barrier()
        pltpu.sync_copy(acc.at[pl.ds(s * rps, rps)], out_hbm.at[c, s])

    return k


def _scatter_sc(a, src4, dst4, ew4):
    n, d = a.shape
    nw, ngr, gch, ce = src4.shape
    k = _make_scatter_sc(n, d, nw, ngr, gch, ce)
    return k(a, src4, dst4, ew4).reshape(_NC, n, d)


def kernel(x, edge_index, edge_attr, number_of_layers, W1_in, b1_in, W2_in,
           W3_in, b3_in, W1_hid, b1_hid, W2_hid, W3_hid, b3_hid, W1_out,
           b1_out, W2_out, W3_out, b3_out, g1, be1, g2, be2):
    n, d = x.shape
    e = edge_attr.shape[0]
    nw = _NC * _NS
    ew_per = e // nw
    ce = 80
    ngr = 5
    gch = -(-ew_per // (ce * ngr))
    epw = ngr * gch * ce

    def shard(a, fill):
        a = a.reshape(nw, ew_per)
        if epw != ew_per:
            a = jnp.pad(a, ((0, 0), (0, epw - ew_per)), constant_values=fill)
        return a.reshape(nw, ngr, gch, ce)

    src4 = shard(edge_index[0], 0)
    dst4 = shard(edge_index[1], 0)
    ew4 = shard(edge_attr, 0.0)

    z = _standardize_tc(x)
    wparts = _wdeg_sc(edge_index[1].reshape(nw, ew_per),
                      edge_attr.reshape(nw, ew_per), n)
    wdeg = _wsum_tc(wparts)

    def layer(h, w1, b1, w2, w3, b3, g, be, final):
        a, bm, cm = _mm3_tc(h, w1, b1, w2, w3, b3)
        parts = _scatter_sc(a, src4, dst4, ew4)
        return _combine_tc(parts, bm, cm, wdeg, g, be, final)

    h = layer(z, W1_in, b1_in, W2_in, W3_in, b3_in, g1, be1, False)
    h = lax.fori_loop(
        0, number_of_layers - 2,
        lambda _, hc: layer(hc, W1_hid, b1_hid, W2_hid, W3_hid, b3_hid,
                            g2, be2, False),
        h)
    return layer(h, W1_out, b1_out, W2_out, W3_out, b3_out, g1, be1, True)


# speedup vs baseline: 1.4218x; 1.0177x over previous
import functools

import jax
import jax.numpy as jnp
from jax import lax
from jax.experimental import pallas as pl
from jax.experimental.pallas import tpu as pltpu
from jax.experimental.pallas import tpu_sc as plsc

_NC = 2
_NS = 16
_NL = 16


def _standardize_tc(x):
    n = x.shape[0]

    def body(x_ref, o_ref):
        xv = x_ref[...]
        mu = jnp.mean(xv, axis=0, keepdims=True)
        var = jnp.sum((xv - mu) ** 2, axis=0, keepdims=True) / (n - 1)
        o_ref[...] = (xv - mu) / (jnp.sqrt(var) + 1e-6)

    return pl.pallas_call(
        body, out_shape=jax.ShapeDtypeStruct(x.shape, x.dtype))(x)


def _mm3_tc(h, w1, b1, w2, w3, b3):
    n, d = h.shape
    o = w1.shape[1]

    def body(h_ref, w1_ref, b1_ref, w2_ref, w3_ref, b3_ref, a_ref, b_ref, c_ref):
        hv = h_ref[...]
        a_ref[...] = jnp.dot(hv, w1_ref[...], preferred_element_type=jnp.float32) + b1_ref[...]
        b_ref[...] = jnp.dot(hv, w2_ref[...], preferred_element_type=jnp.float32)
        c_ref[...] = jnp.dot(hv, w3_ref[...], preferred_element_type=jnp.float32) + b3_ref[...]

    outs = (jax.ShapeDtypeStruct((n, o), jnp.float32),) * 3
    return pl.pallas_call(body, out_shape=outs)(h, w1, b1, w2, w3, b3)


def _wsum_tc(wparts):
    n = wparts.shape[1]

    def body(p_ref, o_ref):
        o_ref[...] = jnp.sum(p_ref[...], axis=0)[:, None]

    return pl.pallas_call(
        body, out_shape=jax.ShapeDtypeStruct((n, 1), jnp.float32))(wparts)


def _combine_tc(parts, bm, cm, wdeg, g, be, final):
    n, d = bm.shape

    def body(p_ref, b_ref, c_ref, w_ref, g_ref, be_ref, o_ref):
        v = p_ref[0] + p_ref[1] - b_ref[...] * w_ref[...] + c_ref[...]
        if final:
            o_ref[...] = jax.nn.sigmoid(v)
        else:
            mu = jnp.mean(v, axis=1, keepdims=True)
            var = jnp.mean((v - mu) ** 2, axis=1, keepdims=True)
            y = (v - mu) / jnp.sqrt(var + 1e-5) * g_ref[...] + be_ref[...]
            o_ref[...] = jnp.where(y >= 0, y, 0.1 * y)

    return pl.pallas_call(
        body, out_shape=jax.ShapeDtypeStruct((n, d), jnp.float32))(
            parts, bm, cm, wdeg, g, be)


def _wdeg_sc(dst2, ew2, n):
    nw = _NC * _NS
    ew_per = dst2.shape[1]
    mesh = plsc.VectorSubcoreMesh(core_axis_name="c", subcore_axis_name="s")

    @functools.partial(
        pl.kernel,
        out_type=jax.ShapeDtypeStruct((nw, n), jnp.float32),
        mesh=mesh,
        compiler_params=pltpu.CompilerParams(needs_layout_passes=False),
        scratch_types=[
            pltpu.VMEM((ew_per,), jnp.int32),
            pltpu.VMEM((ew_per,), jnp.float32),
            pltpu.VMEM((n,), jnp.float32),
        ],
    )
    def k(dst_hbm, ew_hbm, out_hbm, dstv, ewv, wacc):
        c = lax.axis_index("c")
        s = lax.axis_index("s")
        wid = s * _NC + c

        def zero(i, carry):
            wacc[pl.ds(i * _NL, _NL)] = jnp.zeros((_NL,), jnp.float32)
            return carry

        lax.fori_loop(0, n // _NL, zero, 0)
        pltpu.sync_copy(dst_hbm.at[wid], dstv)
        pltpu.sync_copy(ew_hbm.at[wid], ewv)

        def accum(i, carry):
            dv = dstv[pl.ds(i * _NL, _NL)]
            wv = ewv[pl.ds(i * _NL, _NL)]
            plsc.addupdate_scatter(wacc, [dv], wv)
            return carry

        lax.fori_loop(0, ew_per // _NL, accum, 0)
        pltpu.sync_copy(wacc, out_hbm.at[wid])

    return k(dst2, ew2)


@functools.lru_cache(maxsize=None)
def _make_scatter_sc(n, d, nw, ngr, gch, ce):
    rps = n // _NS
    zr = 25
    while rps % zr:
        zr -= 1
    nq = (gch + 1 + 1) // 2
    mesh = plsc.VectorSubcoreMesh(core_axis_name="c", subcore_axis_name="s")

    @functools.partial(
        pl.kernel,
        out_type=jax.ShapeDtypeStruct((_NC, _NS, rps, d), jnp.float32),
        mesh=mesh,
        compiler_params=pltpu.CompilerParams(needs_layout_passes=False),
        scratch_types=[
            pltpu.VMEM((3 * gch, ce), jnp.int32),
            pltpu.VMEM((ce, d), jnp.float32),
            pltpu.VMEM((ce, d), jnp.float32),
            pltpu.VMEM((zr, d), jnp.float32),
            pltpu.VMEM_SHARED((n, d), jnp.float32),
            pltpu.SemaphoreType.DMA,
            pltpu.SemaphoreType.DMA,
            pltpu.SemaphoreType.DMA,
            pltpu.SemaphoreType.DMA,
        ],
    )
    def k(a_hbm, edges_hbm, out_hbm, stg, r0, r1, zbuf, acc, g0, g1, s0, s1):
        rows = (r0, r1)
        gsem = (g0, g1)
        ssem = (s0, s1)
        c = lax.axis_index("c")
        s = lax.axis_index("s")
        wid = s * _NC + c

        def zrow(i, carry):
            for j in range(d // _NL):
                zbuf[i, pl.ds(j * _NL, _NL)] = jnp.zeros((_NL,), jnp.float32)
            return carry

        lax.fori_loop(0, zr, zrow, 0)
        for r in range(rps // zr):
            pltpu.async_copy(zbuf, acc.at[pl.ds(s * rps + r * zr, zr)], g0)
        for r in range(rps // zr):
            pltpu.make_async_copy(
                zbuf, acc.at[pl.ds(s * rps + r * zr, zr)], g0).wait()
        plsc.subcore_barrier()

        def group(g, carry):
            pltpu.sync_copy(edges_hbm.at[wid, g], stg)

            def pair(q, c2):
                for b in range(2):
                    p = q * 2 + b
                    b1 = (b + 1) % 2

                    @pl.when(jnp.logical_and(p >= 2, p < gch))
                    def _():
                        pltpu.make_async_copy(
                            rows[b], acc.at[stg.at[gch + p - 2]], ssem[b]).wait()

                    @pl.when(p < gch)
                    def _():
                        pltpu.async_copy(
                            a_hbm.at[stg.at[p]], rows[b], gsem[b])

                    @pl.when(jnp.logical_and(p >= 1, p < gch + 1))
                    def _():
                        kk = p - 1
                        pltpu.make_async_copy(
                            a_hbm.at[stg.at[kk]], rows[b1], gsem[b1]).wait()

                        def vec16(gg, c3):
                            wv = plsc.bitcast(
                                stg[2 * gch + kk, pl.ds(gg * _NL, _NL)], jnp.float32)
                            for t in range(_NL):
                                w = wv[t]
                                e = gg * _NL + t
                                for j in range(d // _NL):
                                    sl = pl.ds(j * _NL, _NL)
                                    rows[b1][e, sl] = rows[b1][e, sl] * w
                            return c3

                        lax.fori_loop(0, ce // _NL, vec16, 0)
                        pltpu.async_copy(
                            rows[b1], acc.at[stg.at[gch + kk]], ssem[b1], add=True)
                return c2

            lax.fori_loop(0, nq, pair, 0)
            for b in range(2):
                kk = gch - 2 + ((b - (gch - 2)) % 2)
                pltpu.make_async_copy(
                    rows[b], acc.at[stg.at[gch + kk]], ssem[b]).wait()
            return carry

        lax.fori_loop(0, ngr, group, 0)
        plsc.subcore_barrier()
        pltpu.sync_copy(acc.at[pl.ds(s * rps, rps)], out_hbm.at[c, s])

    return k


def _scatter_sc(a, edges):
    n, d = a.shape
    nw, ngr, gch3, ce = edges.shape
    gch = gch3 // 3
    k = _make_scatter_sc(n, d, nw, ngr, gch, ce)
    return k(a, edges).reshape(_NC, n, d)


def kernel(x, edge_index, edge_attr, number_of_layers, W1_in, b1_in, W2_in,
           W3_in, b3_in, W1_hid, b1_hid, W2_hid, W3_hid, b3_hid, W1_out,
           b1_out, W2_out, W3_out, b3_out, g1, be1, g2, be2):
    n, d = x.shape
    e = edge_attr.shape[0]
    nw = _NC * _NS
    ew_per = e // nw
    ce = 80
    ngr = 5
    gch = -(-ew_per // (ce * ngr))
    epw = ngr * gch * ce

    def shard(a, fill):
        a = a.reshape(nw, ew_per)
        if epw != ew_per:
            a = jnp.pad(a, ((0, 0), (0, epw - ew_per)), constant_values=fill)
        return a.reshape(nw, ngr, gch, ce)

    edges = jnp.concatenate(
        [shard(edge_index[0], 0),
         shard(edge_index[1], 0),
         shard(jax.lax.bitcast_convert_type(edge_attr, jnp.int32), 0)],
        axis=2)

    z = _standardize_tc(x)
    wparts = _wdeg_sc(edge_index[1].reshape(nw, ew_per),
                      edge_attr.reshape(nw, ew_per), n)
    wdeg = _wsum_tc(wparts)

    def layer(h, w1, b1, w2, w3, b3, g, be, final):
        a, bm, cm = _mm3_tc(h, w1, b1, w2, w3, b3)
        parts = _scatter_sc(a, edges)
        return _combine_tc(parts, bm, cm, wdeg, g, be, final)

    h = layer(z, W1_in, b1_in, W2_in, W3_in, b3_in, g1, be1, False)
    h = lax.fori_loop(
        0, number_of_layers - 2,
        lambda _, hc: layer(hc, W1_hid, b1_hid, W2_hid, W3_hid, b3_hid,
                            g2, be2, False),
        h)
    return layer(h, W1_out, b1_out, W2_out, W3_out, b3_out, g1, be1, True)
